# baseline (speedup 1.0000x reference)
import functools

import jax
import jax.numpy as jnp
from jax import lax
from jax.experimental import pallas as pl
from jax.experimental.pallas import tpu as pltpu
from jax.experimental.pallas import tpu_sc as plsc

_SQRT2 = 1.4142135623730951
_INV_SQRT2 = 1.0 / _SQRT2

_B = 2048
_N = 4096
_T = _N // 2
_NW = 32
_RPW = _B // _NW
_RB = 4


def _sc_body(x_hbm, out_hbm, xbuf0, xbuf1, ca1, ca2, c0b, c1b, c2b, c3b,
             obuf0, obuf1, isem0, isem1, osem0, osem1):
    wid = lax.axis_index("s") * 2 + lax.axis_index("c")
    base = wid * _RPW
    iota = lax.iota(jnp.int32, 16)
    inv = jnp.full((16,), _INV_SQRT2, jnp.float32)
    ninf = jnp.full((16,), -jnp.inf, jnp.float32)
    zero = jnp.zeros((16,), jnp.float32)

    def masks(ref_, t0):
        c = ref_[pl.ds(t0, 16)]
        cn = plsc.load_gather(ref_, [t0 + 1 + iota])
        sb = plsc.bitcast(c, jnp.int32) < 0
        sbn = plsc.bitcast(cn, jnp.int32) < 0
        up = jnp.logical_and(sbn, jnp.logical_not(sb))
        dn = jnp.logical_and(sb, jnp.logical_not(sbn))
        return c, up, dn

    def compute_row(xb, ob, rb):
        rbv = jnp.full((16,), rb, jnp.int32)

        @plsc.parallel_loop(0, _T // 16, unroll=4)
        def l1(i):
            idx = i * 32 + iota * 2
            e = plsc.load_gather(xb, [rbv, idx])
            o = plsc.load_gather(xb, [rbv, idx + 1])
            ca1[pl.ds(i * 16, 16)] = (e + o) * inv
            c3b[pl.ds(i * 16, 16)] = (e - o) * inv

        @plsc.parallel_loop(0, _T // 32, unroll=4, carry=ninf)
        def m2v(i, m):
            idx = i * 32 + iota * 2
            e = plsc.load_gather(ca1, [idx])
            o = plsc.load_gather(ca1, [idx + 1])
            d = (e - o) * inv
            ca2[pl.ds(i * 16, 16)] = (e + o) * inv
            c2b[pl.ds(i * 16, 16)] = d
            return jnp.maximum(m, d)

        @plsc.parallel_loop(0, _T // 64, unroll=4, carry=(ninf, ninf))
        def m01v(i, ms):
            m0, m1 = ms
            idx = i * 32 + iota * 2
            e = plsc.load_gather(ca2, [idx])
            o = plsc.load_gather(ca2, [idx + 1])
            a = (e + o) * inv
            d = (e - o) * inv
            c0b[pl.ds(i * 16, 16)] = a
            c1b[pl.ds(i * 16, 16)] = d
            return jnp.maximum(m0, a), jnp.maximum(m1, d)

        mx0 = lax.reduce_max(m01v[0], (0,))
        mx1 = lax.reduce_max(m01v[1], (0,))
        mx2 = lax.reduce_max(m2v, (0,))
        sp0 = jnp.full((16,), 1.0, jnp.float32) * mx0
        sp1 = jnp.full((16,), 1.0, jnp.float32) * mx1
        sp2 = jnp.full((16,), 1.0, jnp.float32) * mx2
        c0b[pl.ds(_T // 4, 16)] = sp0
        c1b[pl.ds(_T // 4, 16)] = sp1
        c2b[pl.ds(_T // 2, 16)] = sp2
        last = plsc.load_gather(c3b, [jnp.full((16,), _T - 1, jnp.int32)])
        c3b[pl.ds(_T, 16)] = last

        @plsc.parallel_loop(0, _T // 64, unroll=2)
        def rega(i):
            t0 = i * 16
            c0, u0, _ = masks(c0b, t0)
            c1, u1, d1 = masks(c1b, t0)
            c2, u2, d2 = masks(c2b, t0)
            c3, _, d3 = masks(c3b, t0)
            ob[rb, 0, pl.ds(t0, 16)] = (jnp.where(u0, zero, c0)
                                      + jnp.where(d1, c1, zero))
            ob[rb, 1, pl.ds(t0, 16)] = (jnp.where(u0, c0, zero)
                                      + jnp.where(jnp.logical_or(u1, d1), zero, c1)
                                      + jnp.where(d2, c2, zero))
            ob[rb, 2, pl.ds(t0, 16)] = (jnp.where(u1, c1, zero)
                                      + jnp.where(jnp.logical_or(u2, d2), zero, c2)
                                      + jnp.where(d3, c3, zero))
            ob[rb, 3, pl.ds(t0, 16)] = (jnp.where(u2, c2, zero)
                                      + jnp.where(d3, zero, c3))

        @plsc.parallel_loop(_T // 64, _T // 32, unroll=2)
        def regb(i):
            t0 = i * 16
            c2, u2, d2 = masks(c2b, t0)
            c3, _, d3 = masks(c3b, t0)
            ob[rb, 0, pl.ds(t0, 16)] = sp0
            ob[rb, 1, pl.ds(t0, 16)] = sp1 + jnp.where(d2, c2, zero)
            ob[rb, 2, pl.ds(t0, 16)] = (jnp.where(jnp.logical_or(u2, d2), zero, c2)
                                      + jnp.where(d3, c3, zero))
            ob[rb, 3, pl.ds(t0, 16)] = (jnp.where(u2, c2, zero)
                                      + jnp.where(d3, zero, c3))

        @plsc.parallel_loop(_T // 32, _T // 16, unroll=8)
        def regc(i):
            t0 = i * 16
            c3, _, d3 = masks(c3b, t0)
            ob[rb, 0, pl.ds(t0, 16)] = sp0
            ob[rb, 1, pl.ds(t0, 16)] = sp1
            ob[rb, 2, pl.ds(t0, 16)] = sp2 + jnp.where(d3, c3, zero)
            ob[rb, 3, pl.ds(t0, 16)] = jnp.where(d3, zero, c3)


    nblk = _RPW // _RB
    pltpu.make_async_copy(x_hbm.at[pl.ds(base, _RB)], xbuf0, isem0).start()

    def per_pair(gg, carry):
        for ph, xb, xb_n, ob, isem, isem_n, osem in (
                (0, xbuf0, xbuf1, obuf0, isem0, isem1, osem0),
                (1, xbuf1, xbuf0, obuf1, isem1, isem0, osem1)):
            blk = gg * 2 + ph
            row0 = base + blk * _RB
            pltpu.make_async_copy(x_hbm.at[pl.ds(row0, _RB)], xb, isem).wait()

            @pl.when(blk < nblk - 1)
            def _():
                pltpu.make_async_copy(
                    x_hbm.at[pl.ds(row0 + _RB, _RB)], xb_n, isem_n).start()

            @pl.when(blk >= 2)
            def _():
                pltpu.make_async_copy(
                    ob, out_hbm.at[pl.ds(row0 - 2 * _RB, _RB)], osem).wait()

            for rb in range(_RB):
                compute_row(xb, ob, rb)
            pltpu.make_async_copy(
                ob, out_hbm.at[pl.ds(row0, _RB)], osem).start()
        return carry

    lax.fori_loop(0, nblk // 2, per_pair, 0)
    pltpu.make_async_copy(
        obuf0, out_hbm.at[pl.ds(base + _RPW - 2 * _RB, _RB)], osem0).wait()
    pltpu.make_async_copy(
        obuf1, out_hbm.at[pl.ds(base + _RPW - _RB, _RB)], osem1).wait()


@functools.partial(jax.jit, static_argnums=())
def _sst_sc(x):
    mesh = plsc.VectorSubcoreMesh(core_axis_name="c", subcore_axis_name="s")
    run = functools.partial(
        pl.kernel,
        mesh=mesh,
        out_type=jax.ShapeDtypeStruct((_B, 4, _T), jnp.float32),
        compiler_params=pltpu.CompilerParams(needs_layout_passes=False),
        scratch_types=[
            pltpu.VMEM((_RB, _N), jnp.float32),
            pltpu.VMEM((_RB, _N), jnp.float32),
            pltpu.VMEM((_T,), jnp.float32),
            pltpu.VMEM((_T // 2,), jnp.float32),
            pltpu.VMEM((_T // 4 + 16,), jnp.float32),
            pltpu.VMEM((_T // 4 + 16,), jnp.float32),
            pltpu.VMEM((_T // 2 + 16,), jnp.float32),
            pltpu.VMEM((_T + 16,), jnp.float32),
            pltpu.VMEM((_RB, 4, _T), jnp.float32),
            pltpu.VMEM((_RB, 4, _T), jnp.float32),
            pltpu.SemaphoreType.DMA,
            pltpu.SemaphoreType.DMA,
            pltpu.SemaphoreType.DMA,
            pltpu.SemaphoreType.DMA,
        ],
    )(_sc_body)
    return run(x)


def kernel(x):
    return _sst_sc(x)


# speedup vs baseline: 1.0829x; 1.0829x over previous
import functools

import jax
import jax.numpy as jnp
from jax import lax
from jax.experimental import pallas as pl
from jax.experimental.pallas import tpu as pltpu
from jax.experimental.pallas import tpu_sc as plsc

_SQRT2 = 1.4142135623730951
_INV_SQRT2 = 1.0 / _SQRT2

_B = 2048
_N = 4096
_T = _N // 2
_NW = 32
_RPW = _B // _NW
_RB = 2


def _sc_body(x_hbm, out_hbm, xbuf0, xbuf1, ca1, ca2, c0b, c1b, c2b, c3b,
             obuf0, obuf1, isem0, isem1, osem0, osem1):
    wid = lax.axis_index("s") * 2 + lax.axis_index("c")
    base = wid * _RPW
    iota = lax.iota(jnp.int32, 16)
    inv = jnp.full((16,), _INV_SQRT2, jnp.float32)
    ninf = jnp.full((16,), -jnp.inf, jnp.float32)
    zero = jnp.zeros((16,), jnp.float32)

    def masks(ref_, t0):
        c = ref_[pl.ds(t0, 16)]
        cn = plsc.load_gather(ref_, [t0 + 1 + iota])
        sb = plsc.bitcast(c, jnp.int32) < 0
        sbn = plsc.bitcast(cn, jnp.int32) < 0
        up = jnp.logical_and(sbn, jnp.logical_not(sb))
        dn = jnp.logical_and(sb, jnp.logical_not(sbn))
        return c, up, dn

    def compute_row(xb, ob, rb):
        rbv = jnp.full((16,), rb, jnp.int32)

        @plsc.parallel_loop(0, _T // 16, unroll=4)
        def l1(i):
            idx = i * 32 + iota * 2
            e = plsc.load_gather(xb, [rbv, idx])
            o = plsc.load_gather(xb, [rbv, idx + 1])
            ca1[pl.ds(i * 16, 16)] = (e + o) * inv
            c3b[pl.ds(i * 16, 16)] = (e - o) * inv

        @plsc.parallel_loop(0, _T // 32, unroll=4, carry=ninf)
        def m2v(i, m):
            idx = i * 32 + iota * 2
            e = plsc.load_gather(ca1, [idx])
            o = plsc.load_gather(ca1, [idx + 1])
            d = (e - o) * inv
            ca2[pl.ds(i * 16, 16)] = (e + o) * inv
            c2b[pl.ds(i * 16, 16)] = d
            return jnp.maximum(m, d)

        @plsc.parallel_loop(0, _T // 64, unroll=4, carry=(ninf, ninf))
        def m01v(i, ms):
            m0, m1 = ms
            idx = i * 32 + iota * 2
            e = plsc.load_gather(ca2, [idx])
            o = plsc.load_gather(ca2, [idx + 1])
            a = (e + o) * inv
            d = (e - o) * inv
            c0b[pl.ds(i * 16, 16)] = a
            c1b[pl.ds(i * 16, 16)] = d
            return jnp.maximum(m0, a), jnp.maximum(m1, d)

        mx0 = lax.reduce_max(m01v[0], (0,))
        mx1 = lax.reduce_max(m01v[1], (0,))
        mx2 = lax.reduce_max(m2v, (0,))
        sp0 = jnp.full((16,), 1.0, jnp.float32) * mx0
        sp1 = jnp.full((16,), 1.0, jnp.float32) * mx1
        sp2 = jnp.full((16,), 1.0, jnp.float32) * mx2
        c0b[pl.ds(_T // 4, 16)] = sp0
        c1b[pl.ds(_T // 4, 16)] = sp1
        c2b[pl.ds(_T // 2, 16)] = sp2
        last = plsc.load_gather(c3b, [jnp.full((16,), _T - 1, jnp.int32)])
        c3b[pl.ds(_T, 16)] = last

        @plsc.parallel_loop(0, _T // 64, unroll=2)
        def rega(i):
            t0 = i * 16
            c0, u0, _ = masks(c0b, t0)
            c1, u1, d1 = masks(c1b, t0)
            c2, u2, d2 = masks(c2b, t0)
            c3, _, d3 = masks(c3b, t0)
            ob[rb, 0, pl.ds(t0, 16)] = (jnp.where(u0, zero, c0)
                                      + jnp.where(d1, c1, zero))
            ob[rb, 1, pl.ds(t0, 16)] = (jnp.where(u0, c0, zero)
                                      + jnp.where(jnp.logical_or(u1, d1), zero, c1)
                                      + jnp.where(d2, c2, zero))
            ob[rb, 2, pl.ds(t0, 16)] = (jnp.where(u1, c1, zero)
                                      + jnp.where(jnp.logical_or(u2, d2), zero, c2)
                                      + jnp.where(d3, c3, zero))
            ob[rb, 3, pl.ds(t0, 16)] = (jnp.where(u2, c2, zero)
                                      + jnp.where(d3, zero, c3))

        @plsc.parallel_loop(_T // 64, _T // 32, unroll=2)
        def regb(i):
            t0 = i * 16
            c2, u2, d2 = masks(c2b, t0)
            c3, _, d3 = masks(c3b, t0)
            ob[rb, 0, pl.ds(t0, 16)] = sp0
            ob[rb, 1, pl.ds(t0, 16)] = sp1 + jnp.where(d2, c2, zero)
            ob[rb, 2, pl.ds(t0, 16)] = (jnp.where(jnp.logical_or(u2, d2), zero, c2)
                                      + jnp.where(d3, c3, zero))
            ob[rb, 3, pl.ds(t0, 16)] = (jnp.where(u2, c2, zero)
                                      + jnp.where(d3, zero, c3))

        @plsc.parallel_loop(_T // 32, _T // 16, unroll=8)
        def regc(i):
            t0 = i * 16
            c3, _, d3 = masks(c3b, t0)
            ob[rb, 0, pl.ds(t0, 16)] = sp0
            ob[rb, 1, pl.ds(t0, 16)] = sp1
            ob[rb, 2, pl.ds(t0, 16)] = sp2 + jnp.where(d3, c3, zero)
            ob[rb, 3, pl.ds(t0, 16)] = jnp.where(d3, zero, c3)


    nblk = _RPW // _RB
    pltpu.make_async_copy(x_hbm.at[pl.ds(base, _RB)], xbuf0, isem0).start()

    def per_pair(gg, carry):
        for ph, xb, xb_n, ob, isem, isem_n, osem in (
                (0, xbuf0, xbuf1, obuf0, isem0, isem1, osem0),
                (1, xbuf1, xbuf0, obuf1, isem1, isem0, osem1)):
            blk = gg * 2 + ph
            row0 = base + blk * _RB
            pltpu.make_async_copy(x_hbm.at[pl.ds(row0, _RB)], xb, isem).wait()

            @pl.when(blk < nblk - 1)
            def _():
                pltpu.make_async_copy(
                    x_hbm.at[pl.ds(row0 + _RB, _RB)], xb_n, isem_n).start()

            @pl.when(blk >= 2)
            def _():
                pltpu.make_async_copy(
                    ob, out_hbm.at[pl.ds(row0 - 2 * _RB, _RB)], osem).wait()

            for rb in range(_RB):
                compute_row(xb, ob, rb)
            pltpu.make_async_copy(
                ob, out_hbm.at[pl.ds(row0, _RB)], osem).start()
        return carry

    lax.fori_loop(0, nblk // 2, per_pair, 0)
    pltpu.make_async_copy(
        obuf0, out_hbm.at[pl.ds(base + _RPW - 2 * _RB, _RB)], osem0).wait()
    pltpu.make_async_copy(
        obuf1, out_hbm.at[pl.ds(base + _RPW - _RB, _RB)], osem1).wait()


@functools.partial(jax.jit, static_argnums=())
def _sst_sc(x):
    mesh = plsc.VectorSubcoreMesh(core_axis_name="c", subcore_axis_name="s")
    run = functools.partial(
        pl.kernel,
        mesh=mesh,
        out_type=jax.ShapeDtypeStruct((_B, 4, _T), jnp.float32),
        compiler_params=pltpu.CompilerParams(needs_layout_passes=False),
        scratch_types=[
            pltpu.VMEM((_RB, _N), jnp.float32),
            pltpu.VMEM((_RB, _N), jnp.float32),
            pltpu.VMEM((_T,), jnp.float32),
            pltpu.VMEM((_T // 2,), jnp.float32),
            pltpu.VMEM((_T // 4 + 16,), jnp.float32),
            pltpu.VMEM((_T // 4 + 16,), jnp.float32),
            pltpu.VMEM((_T // 2 + 16,), jnp.float32),
            pltpu.VMEM((_T + 16,), jnp.float32),
            pltpu.VMEM((_RB, 4, _T), jnp.float32),
            pltpu.VMEM((_RB, 4, _T), jnp.float32),
            pltpu.SemaphoreType.DMA,
            pltpu.SemaphoreType.DMA,
            pltpu.SemaphoreType.DMA,
            pltpu.SemaphoreType.DMA,
        ],
    )(_sc_body)
    return run(x)


def kernel(x):
    return _sst_sc(x)


# speedup vs baseline: 1.0841x; 1.0011x over previous
import functools

import jax
import jax.numpy as jnp
from jax import lax
from jax.experimental import pallas as pl
from jax.experimental.pallas import tpu as pltpu
from jax.experimental.pallas import tpu_sc as plsc

_SQRT2 = 1.4142135623730951
_INV_SQRT2 = 1.0 / _SQRT2

_B = 2048
_N = 4096
_T = _N // 2
_NW = 32
_RPW = _B // _NW
_RB = 2


def _sc_body(x_hbm, out_hbm, xbuf0, xbuf1, ca1, ca2, c0b, c1b, c2b, c3b,
             obuf0, obuf1, isem0, isem1, osem0, osem1):
    wid = lax.axis_index("s") * 2 + lax.axis_index("c")
    base = wid * _RPW
    iota = lax.iota(jnp.int32, 16)
    inv = jnp.full((16,), _INV_SQRT2, jnp.float32)
    ninf = jnp.full((16,), -jnp.inf, jnp.float32)
    zero = jnp.zeros((16,), jnp.float32)

    def masks(ref_, t0):
        c = ref_[pl.ds(t0, 16)]
        cn = plsc.load_gather(ref_, [t0 + 1 + iota])
        sb = plsc.bitcast(c, jnp.int32) < 0
        sbn = plsc.bitcast(cn, jnp.int32) < 0
        up = jnp.logical_and(sbn, jnp.logical_not(sb))
        dn = jnp.logical_and(sb, jnp.logical_not(sbn))
        return c, up, dn

    dnums = lax.GatherDimensionNumbers(
        offset_dims=(), collapsed_slice_dims=(0,), start_index_map=(0,))

    def vperm(a, idx):
        return lax.gather(a, idx[:, None], dnums, (1,),
                          mode=lax.GatherScatterMode.PROMISE_IN_BOUNDS)

    idx_e = (iota * 2) % 16
    idx_o = idx_e + 1
    lane_lo = iota < 8

    def compute_row(xb, ob, rb):
        rbv = jnp.full((16,), rb, jnp.int32)

        @plsc.parallel_loop(0, _T // 32, unroll=4, carry=ninf)
        def m2v(i, m):
            idx = i * 64 + iota * 2
            e_lo = plsc.load_gather(xb, [rbv, idx])
            o_lo = plsc.load_gather(xb, [rbv, idx + 1])
            e_hi = plsc.load_gather(xb, [rbv, idx + 32])
            o_hi = plsc.load_gather(xb, [rbv, idx + 33])
            c3b[pl.ds(i * 32, 16)] = (e_lo - o_lo) * inv
            c3b[pl.ds(i * 32 + 16, 16)] = (e_hi - o_hi) * inv
            a_lo = (e_lo + o_lo) * inv
            a_hi = (e_hi + o_hi) * inv
            ev = jnp.where(lane_lo, vperm(a_lo, idx_e), vperm(a_hi, idx_e))
            od = jnp.where(lane_lo, vperm(a_lo, idx_o), vperm(a_hi, idx_o))
            d = (ev - od) * inv
            ca2[pl.ds(i * 16, 16)] = (ev + od) * inv
            c2b[pl.ds(i * 16, 16)] = d
            return jnp.maximum(m, d)

        @plsc.parallel_loop(0, _T // 64, unroll=4, carry=(ninf, ninf))
        def m01v(i, ms):
            m0, m1 = ms
            idx = i * 32 + iota * 2
            e = plsc.load_gather(ca2, [idx])
            o = plsc.load_gather(ca2, [idx + 1])
            a = (e + o) * inv
            d = (e - o) * inv
            c0b[pl.ds(i * 16, 16)] = a
            c1b[pl.ds(i * 16, 16)] = d
            return jnp.maximum(m0, a), jnp.maximum(m1, d)

        mx0 = lax.reduce_max(m01v[0], (0,))
        mx1 = lax.reduce_max(m01v[1], (0,))
        mx2 = lax.reduce_max(m2v, (0,))
        sp0 = jnp.full((16,), 1.0, jnp.float32) * mx0
        sp1 = jnp.full((16,), 1.0, jnp.float32) * mx1
        sp2 = jnp.full((16,), 1.0, jnp.float32) * mx2
        c0b[pl.ds(_T // 4, 16)] = sp0
        c1b[pl.ds(_T // 4, 16)] = sp1
        c2b[pl.ds(_T // 2, 16)] = sp2
        last = plsc.load_gather(c3b, [jnp.full((16,), _T - 1, jnp.int32)])
        c3b[pl.ds(_T, 16)] = last

        @plsc.parallel_loop(0, _T // 64, unroll=2)
        def rega(i):
            t0 = i * 16
            c0, u0, _ = masks(c0b, t0)
            c1, u1, d1 = masks(c1b, t0)
            c2, u2, d2 = masks(c2b, t0)
            c3, _, d3 = masks(c3b, t0)
            ob[rb, 0, pl.ds(t0, 16)] = (jnp.where(u0, zero, c0)
                                      + jnp.where(d1, c1, zero))
            ob[rb, 1, pl.ds(t0, 16)] = (jnp.where(u0, c0, zero)
                                      + jnp.where(jnp.logical_or(u1, d1), zero, c1)
                                      + jnp.where(d2, c2, zero))
            ob[rb, 2, pl.ds(t0, 16)] = (jnp.where(u1, c1, zero)
                                      + jnp.where(jnp.logical_or(u2, d2), zero, c2)
                                      + jnp.where(d3, c3, zero))
            ob[rb, 3, pl.ds(t0, 16)] = (jnp.where(u2, c2, zero)
                                      + jnp.where(d3, zero, c3))

        @plsc.parallel_loop(_T // 64, _T // 32, unroll=2)
        def regb(i):
            t0 = i * 16
            c2, u2, d2 = masks(c2b, t0)
            c3, _, d3 = masks(c3b, t0)
            ob[rb, 0, pl.ds(t0, 16)] = sp0
            ob[rb, 1, pl.ds(t0, 16)] = sp1 + jnp.where(d2, c2, zero)
            ob[rb, 2, pl.ds(t0, 16)] = (jnp.where(jnp.logical_or(u2, d2), zero, c2)
                                      + jnp.where(d3, c3, zero))
            ob[rb, 3, pl.ds(t0, 16)] = (jnp.where(u2, c2, zero)
                                      + jnp.where(d3, zero, c3))

        @plsc.parallel_loop(_T // 32, _T // 16, unroll=8)
        def regc(i):
            t0 = i * 16
            c3, _, d3 = masks(c3b, t0)
            ob[rb, 0, pl.ds(t0, 16)] = sp0
            ob[rb, 1, pl.ds(t0, 16)] = sp1
            ob[rb, 2, pl.ds(t0, 16)] = sp2 + jnp.where(d3, c3, zero)
            ob[rb, 3, pl.ds(t0, 16)] = jnp.where(d3, zero, c3)


    nblk = _RPW // _RB
    pltpu.make_async_copy(x_hbm.at[pl.ds(base, _RB)], xbuf0, isem0).start()

    def per_pair(gg, carry):
        for ph, xb, xb_n, ob, isem, isem_n, osem in (
                (0, xbuf0, xbuf1, obuf0, isem0, isem1, osem0),
                (1, xbuf1, xbuf0, obuf1, isem1, isem0, osem1)):
            blk = gg * 2 + ph
            row0 = base + blk * _RB
            pltpu.make_async_copy(x_hbm.at[pl.ds(row0, _RB)], xb, isem).wait()

            @pl.when(blk < nblk - 1)
            def _():
                pltpu.make_async_copy(
                    x_hbm.at[pl.ds(row0 + _RB, _RB)], xb_n, isem_n).start()

            @pl.when(blk >= 2)
            def _():
                pltpu.make_async_copy(
                    ob, out_hbm.at[pl.ds(row0 - 2 * _RB, _RB)], osem).wait()

            for rb in range(_RB):
                compute_row(xb, ob, rb)
            pltpu.make_async_copy(
                ob, out_hbm.at[pl.ds(row0, _RB)], osem).start()
        return carry

    lax.fori_loop(0, nblk // 2, per_pair, 0)
    pltpu.make_async_copy(
        obuf0, out_hbm.at[pl.ds(base + _RPW - 2 * _RB, _RB)], osem0).wait()
    pltpu.make_async_copy(
        obuf1, out_hbm.at[pl.ds(base + _RPW - _RB, _RB)], osem1).wait()


@functools.partial(jax.jit, static_argnums=())
def _sst_sc(x):
    mesh = plsc.VectorSubcoreMesh(core_axis_name="c", subcore_axis_name="s")
    run = functools.partial(
        pl.kernel,
        mesh=mesh,
        out_type=jax.ShapeDtypeStruct((_B, 4, _T), jnp.float32),
        compiler_params=pltpu.CompilerParams(needs_layout_passes=False),
        scratch_types=[
            pltpu.VMEM((_RB, _N), jnp.float32),
            pltpu.VMEM((_RB, _N), jnp.float32),
            pltpu.VMEM((_T,), jnp.float32),
            pltpu.VMEM((_T // 2,), jnp.float32),
            pltpu.VMEM((_T // 4 + 16,), jnp.float32),
            pltpu.VMEM((_T // 4 + 16,), jnp.float32),
            pltpu.VMEM((_T // 2 + 16,), jnp.float32),
            pltpu.VMEM((_T + 16,), jnp.float32),
            pltpu.VMEM((_RB, 4, _T), jnp.float32),
            pltpu.VMEM((_RB, 4, _T), jnp.float32),
            pltpu.SemaphoreType.DMA,
            pltpu.SemaphoreType.DMA,
            pltpu.SemaphoreType.DMA,
            pltpu.SemaphoreType.DMA,
        ],
    )(_sc_body)
    return run(x)


def kernel(x):
    return _sst_sc(x)
